# Initial kernel scaffold; baseline (speedup 1.0000x reference)
import jax
import jax.numpy as jnp
from jax.experimental import pallas as pl


def kernel(x, Wg, bg, We, be):
    raise NotImplementedError("write your pallas kernel here")


# speedup vs baseline: 2.1508x; 2.1508x over previous
import jax
import jax.numpy as jnp
from jax.experimental import pallas as pl

N = 2048
D = 768
E = 8
K = 2


def _gate_counts_kernel(x_ref, wg_ref, bg_ref, scale_ref):
    x = x_ref[...]
    wg = wg_ref[...]
    logits = jax.lax.dot_general(
        x, wg, (((1,), (1,)), ((), ())), preferred_element_type=jnp.float32
    )
    logits = logits + bg_ref[...]
    idx = jax.lax.broadcasted_iota(jnp.int32, logits.shape, 1)
    m1 = jnp.max(logits, axis=1, keepdims=True)
    i1 = jnp.min(jnp.where(logits == m1, idx, E), axis=1, keepdims=True)
    oh1 = idx == i1
    masked = jnp.where(oh1, -jnp.inf, logits)
    m2 = jnp.max(masked, axis=1, keepdims=True)
    i2 = jnp.min(jnp.where(masked == m2, idx, E), axis=1, keepdims=True)
    oh2 = idx == i2
    cnt = jnp.sum(oh1.astype(jnp.float32) + oh2.astype(jnp.float32), axis=0)
    scale_ref[...] = (cnt / float(N * K)).reshape(1, E)


def _expert_acc_kernel(scale_ref, x_ref, we_ref, be_ref, out_ref):
    e = pl.program_id(0)
    w = we_ref[0]
    z = jax.lax.dot_general(
        x_ref[...], w, (((1,), (1,)), ((), ())), preferred_element_type=jnp.float32
    )
    r = jnp.maximum(z + be_ref[0], 0.0)
    sv = scale_ref[...]
    oh = jax.lax.broadcasted_iota(jnp.int32, (1, E), 1) == e
    s = jnp.sum(jnp.where(oh, sv, 0.0), axis=(0, 1), keepdims=True)
    contrib = r * s

    @pl.when(e == 0)
    def _():
        out_ref[...] = contrib

    @pl.when(e > 0)
    def _():
        out_ref[...] += contrib


def kernel(x, Wg, bg, We, be):
    bg2 = bg.reshape(1, E)
    scale = pl.pallas_call(
        _gate_counts_kernel,
        out_shape=jax.ShapeDtypeStruct((1, E), jnp.float32),
    )(x, Wg, bg2)

    out = pl.pallas_call(
        _expert_acc_kernel,
        grid=(E,),
        in_specs=[
            pl.BlockSpec((1, E), lambda e: (0, 0)),
            pl.BlockSpec((N, D), lambda e: (0, 0)),
            pl.BlockSpec((1, D, D), lambda e: (e, 0, 0)),
            pl.BlockSpec((1, 1, D), lambda e: (e, 0, 0)),
        ],
        out_specs=pl.BlockSpec((N, D), lambda e: (0, 0)),
        out_shape=jax.ShapeDtypeStruct((N, D), jnp.float32),
    )(scale, x, We, be.reshape(E, 1, D))
    return out
